# baseline (speedup 1.0000x reference)
import functools

import jax
import jax.numpy as jnp
from jax import lax
from jax.experimental import pallas as pl
from jax.experimental.pallas import tpu as pltpu
from jax.experimental.pallas import tpu_sc as plsc

B = 1024
T = 200
D = 64
V = 1000000
BT = B * T
NC = 2
NS = 16
NW = NC * NS
LANES = 16
GROUPS = D // LANES

CB = 128
N_BLOCKS = V // CB
V_MAIN = N_BLOCKS * CB
V_TAIL2 = (V - V_MAIN) // 2
MAX_BLK = -(-N_BLOCKS // NW)
TNBUF = 4

B_PER_W = BT // NW
CHUNK = 64
N_CHUNKS = B_PER_W // CHUNK
NBUF = 4
HT = T // 2
POS_STAGE = 128

_mesh = plsc.VectorSubcoreMesh(core_axis_name="c", subcore_axis_name="s")


@functools.partial(
    pl.kernel,
    mesh=_mesh,
    out_type=jax.ShapeDtypeStruct((V // 2, 2 * D), jnp.float32),
    scratch_types=[
        pltpu.VMEM((TNBUF, D, CB), jnp.float32),
        pltpu.VMEM((TNBUF, CB // 2, 2 * D), jnp.float32),
        pltpu.VMEM((V_TAIL2, 2 * D), jnp.float32),
    ] + [pltpu.SemaphoreType.DMA] * (2 * TNBUF),
    compiler_params=pltpu.CompilerParams(
        use_tc_tiling_on_sc=True, needs_layout_passes=False),
)
def _transpose_sc(tokt_hbm, tail_hbm, tokr_hbm, slab_v, dst_v, tail_v,
                  g0, g1, g2, g3, o0, o1, o2, o3):
    gsems = (g0, g1, g2, g3)
    osems = (o0, o1, o2, o3)
    cid = lax.axis_index("c")
    sid = lax.axis_index("s")
    wid = sid * NC + cid

    @pl.when(wid == 0)
    def _():
        pltpu.sync_copy(tail_hbm, tail_v)
        pltpu.sync_copy(tail_v, tokr_hbm.at[pl.ds(V_MAIN // 2, V_TAIL2)])

    iota = lax.iota(jnp.int32, LANES)

    def blk_of(i):
        return wid + i * NW

    def fetch(i, b):
        c0 = pl.multiple_of(blk_of(i) * CB, CB)
        for dt in range(D // 8):
            pltpu.async_copy(
                tokt_hbm.at[pl.ds(dt * 8, 8), pl.ds(c0, CB)],
                slab_v.at[b, pl.ds(dt * 8, 8)], gsems[b])

    def wait_fetch(b):
        for dt in range(D // 8):
            pltpu.make_async_copy(
                tokt_hbm.at[pl.ds(0, 8), pl.ds(0, CB)],
                slab_v.at[b, pl.ds(0, 8)], gsems[b]).wait()

    def put(i, b):
        r0 = pl.multiple_of(blk_of(i) * (CB // 2), 8)
        pltpu.async_copy(
            dst_v.at[b], tokr_hbm.at[pl.ds(r0, CB // 2)], osems[b])

    def wait_put(b):
        pltpu.make_async_copy(
            dst_v.at[b], tokr_hbm.at[pl.ds(0, CB // 2)], osems[b]).wait()

    for b in range(TNBUF - 1):
        fetch(b, b)

    half_iota = lax.shift_right_logical(iota, 1)
    parity64 = (iota & 1) * D
    rows_tg = [half_iota + tg * (LANES // 2) for tg in range(CB // LANES)]

    def step(s, carry):
        i0 = s * TNBUF
        for b in range(TNBUF):
            i = i0 + b
            nxt = i + TNBUF - 1
            bn = (b + TNBUF - 1) % TNBUF

            @pl.when(blk_of(i) < N_BLOCKS)
            def _():
                @pl.when(blk_of(nxt) < N_BLOCKS)
                def _():
                    @pl.when(nxt >= TNBUF)
                    def _():
                        wait_put(bn)
                    fetch(nxt, bn)

                wait_fetch(b)

                @plsc.parallel_loop(0, D, unroll=4)
                def _tp(d):
                    cols = parity64 + d
                    for tg in range(CB // LANES):
                        vec = slab_v[b, d, pl.ds(tg * LANES, LANES)]
                        plsc.store_scatter(
                            dst_v.at[b], [rows_tg[tg], cols], vec)

                put(i, b)
        return carry

    lax.fori_loop(0, -(-MAX_BLK // TNBUF), step, 0)

    for b in range(TNBUF):
        wait_put(b)


@functools.partial(
    pl.kernel,
    mesh=_mesh,
    out_type=jax.ShapeDtypeStruct((BT // 2, 2 * D), jnp.float32),
    scratch_types=[
        pltpu.VMEM((B_PER_W,), jnp.int32),
        pltpu.VMEM((B_PER_W,), jnp.int32),
        pltpu.VMEM((NBUF, CHUNK, 2 * D), jnp.float32),
        pltpu.VMEM((NBUF, CHUNK // 2, 2 * D), jnp.float32),
        pltpu.VMEM((POS_STAGE, 2 * D), jnp.float32),
    ] + [pltpu.SemaphoreType.DMA] * (2 * NBUF),
    compiler_params=pltpu.CompilerParams(use_tc_tiling_on_sc=True),
)
def _embed_sc(x_hbm, tok_hbm, pos_hbm, out_hbm, idx_v, pidx_v, rows_v,
              obuf_v, pos_v, g0, g1, g2, g3, o0, o1, o2, o3):
    gsems = (g0, g1, g2, g3)
    osems = (o0, o1, o2, o3)
    cid = lax.axis_index("c")
    sid = lax.axis_index("s")
    wid = sid * NC + cid
    base = wid * B_PER_W
    base2 = base // 2

    pltpu.sync_copy(x_hbm.at[pl.ds(pl.multiple_of(base, 128), B_PER_W)], idx_v)
    pltpu.sync_copy(pos_hbm.at[pl.ds(0, POS_STAGE)], pos_v)

    @plsc.parallel_loop(0, B_PER_W // LANES, unroll=8)
    def _pidx(q):
        sl = pl.ds(q * LANES, LANES)
        pidx_v[sl] = lax.shift_right_logical(idx_v[sl], 1)

    def gather(ch, b):
        pltpu.async_copy(
            tok_hbm.at[pidx_v.at[pl.ds(pl.multiple_of(ch * CHUNK, 64), CHUNK)]],
            rows_v.at[b], gsems[b])

    def wait_gather(ch, b):
        pltpu.make_async_copy(
            tok_hbm.at[pidx_v.at[pl.ds(pl.multiple_of(ch * CHUNK, 64), CHUNK)]],
            rows_v.at[b], gsems[b]).wait()

    def put(ch, b):
        pltpu.async_copy(
            obuf_v.at[b],
            out_hbm.at[pl.ds(pl.multiple_of(base2 + ch * (CHUNK // 2), 32),
                             CHUNK // 2)],
            osems[b])

    def wait_put(b):
        pltpu.make_async_copy(
            obuf_v.at[b], out_hbm.at[pl.ds(pl.multiple_of(base2, 32),
                                           CHUNK // 2)],
            osems[b]).wait()

    for b in range(NBUF - 1):
        gather(b, b)

    def block_body(blk, carry):
        c0 = blk * NBUF
        for b in range(NBUF):
            ch = c0 + b
            nxt = ch + NBUF - 1
            bn = (b + NBUF - 1) % NBUF

            @pl.when(nxt < N_CHUNKS)
            def _():
                gather(nxt, bn)

            wait_gather(ch, b)

            @pl.when(ch >= NBUF)
            def _():
                wait_put(b)

            p0h = lax.rem(ch * (CHUNK // 2), HT)

            @plsc.parallel_loop(0, CHUNK // LANES, unroll=1)
            def _combine(q):
                par16 = idx_v[pl.ds(ch * CHUNK + q * LANES, LANES)]
                for j in range(LANES):
                    off = (par16[j] & 1) * D
                    r2 = q * (LANES // 2) + j // 2
                    rp = j & 1
                    prow = lax.rem(p0h + r2, HT)
                    for g in range(GROUPS):
                        dsl = pl.ds(rp * D + g * LANES, LANES)
                        ssl = pl.ds(off + g * LANES, LANES)
                        obuf_v[b, r2, dsl] = (
                            rows_v[b, q * LANES + j, ssl]
                            + pos_v[prow, dsl])

            put(ch, b)
        return carry

    lax.fori_loop(0, N_CHUNKS // NBUF, block_body, 0)

    for b in range(NBUF):
        wait_put(b)


def kernel(x, token_emb, pos_emb):
    xflat = x.reshape(BT).astype(jnp.int32)
    tokt = token_emb.T
    tail = token_emb[V_MAIN:].reshape(V_TAIL2, 2 * D)
    pos2 = pos_emb.reshape(-1, 2 * D)
    tokr = _transpose_sc(tokt, tail)
    out = _embed_sc(xflat, tokr, pos2)
    return out.reshape(B, T, D)


# speedup vs baseline: 1.5937x; 1.5937x over previous
import functools

import jax
import jax.numpy as jnp
from jax import lax
from jax.experimental import pallas as pl
from jax.experimental.pallas import tpu as pltpu
from jax.experimental.pallas import tpu_sc as plsc

B = 1024
T = 200
D = 64
BT = B * T
NC = 2
NS = 16
NW = NC * NS
B_PER_W = BT // NW
CHUNK = 64
N_CHUNKS = B_PER_W // CHUNK
NBUF = 4
LANES = 16
GROUPS = D // LANES
POS_PAD = T + CHUNK

_mesh = plsc.VectorSubcoreMesh(core_axis_name="c", subcore_axis_name="s")


@functools.partial(
    pl.kernel,
    mesh=_mesh,
    out_type=jax.ShapeDtypeStruct((BT, D), jnp.float32),
    scratch_types=[
        pltpu.VMEM((N_CHUNKS, CHUNK), jnp.int32),
        pltpu.VMEM((NBUF, CHUNK, D), jnp.float32),
        pltpu.VMEM((POS_PAD, D), jnp.float32),
    ] + [pltpu.SemaphoreType.DMA] * (2 * NBUF),
    compiler_params=pltpu.CompilerParams(use_tc_tiling_on_sc=False),
)
def _embed_sc(x_hbm, tok_hbm, pos_hbm, out_hbm, idx_v, rows_v, pos_v,
              g0, g1, g2, g3, o0, o1, o2, o3):
    gsems = (g0, g1, g2, g3)
    osems = (o0, o1, o2, o3)
    cid = lax.axis_index("c")
    sid = lax.axis_index("s")
    wid = sid * NC + cid
    base = wid * B_PER_W

    pltpu.sync_copy(x_hbm.at[wid], idx_v)
    pltpu.sync_copy(pos_hbm.at[pl.ds(0, T)], pos_v.at[pl.ds(0, T)])
    pltpu.sync_copy(pos_hbm.at[pl.ds(0, CHUNK)], pos_v.at[pl.ds(T, CHUNK)])

    def gather(ch, b):
        pltpu.async_copy(tok_hbm.at[idx_v.at[ch]], rows_v.at[b], gsems[b])

    def wait_gather(ch, b):
        pltpu.make_async_copy(
            tok_hbm.at[idx_v.at[ch]], rows_v.at[b], gsems[b]).wait()

    def put(ch, b):
        pltpu.async_copy(
            rows_v.at[b], out_hbm.at[pl.ds(base + ch * CHUNK, CHUNK)],
            osems[b])

    def wait_put(b):
        pltpu.make_async_copy(
            rows_v.at[b], out_hbm.at[pl.ds(base, CHUNK)], osems[b]).wait()

    for b in range(NBUF - 1):
        gather(b, b)

    def block_body(blk, carry):
        c0 = blk * NBUF
        for b in range(NBUF):
            ch = c0 + b
            nxt = ch + NBUF - 1
            bn = (b + NBUF - 1) % NBUF

            @pl.when(nxt < N_CHUNKS)
            def _():
                @pl.when(nxt >= NBUF)
                def _():
                    wait_put(bn)
                gather(nxt, bn)

            wait_gather(ch, b)
            p0 = lax.rem(ch * CHUNK, T)

            @plsc.parallel_loop(0, CHUNK, unroll=8)
            def _row(r):
                for g in range(GROUPS):
                    sl = pl.ds(g * LANES, LANES)
                    rows_v[b, r, sl] = rows_v[b, r, sl] + pos_v[p0 + r, sl]

            put(ch, b)
        return carry

    lax.fori_loop(0, N_CHUNKS // NBUF, block_body, 0)

    for b in range(NBUF):
        wait_put(b)


def kernel(x, token_emb, pos_emb):
    xw = x.reshape(NW, N_CHUNKS, CHUNK).astype(jnp.int32)
    out = _embed_sc(xw, token_emb, pos_emb)
    return out.reshape(B, T, D)
